# baseline (speedup 1.0000x reference)
import functools

import jax
import jax.numpy as jnp
import numpy as np
from jax import lax
from jax.experimental import pallas as pl

DIM = 256
HEADS = 8
HD = DIM // HEADS
WIN = 8
NE = 8
SCALE = HD ** -0.5
RHID = 128

NTOK = 2 * 64 * 64
TBLK = 512
NTB = NTOK // TBLK


def _rope_tables():
    d = HD // 2
    half = d // 2
    inv = 1.0 / (10000.0 ** (np.arange(half, dtype=np.float64) / half))
    hpos = np.repeat(np.arange(WIN), WIN).astype(np.float64)
    wpos = np.tile(np.arange(WIN), WIN).astype(np.float64)
    ah = hpos[:, None] * inv[None, :]
    aw = wpos[:, None] * inv[None, :]
    cos = np.concatenate([np.cos(ah), np.cos(ah), np.cos(aw), np.cos(aw)],
                         axis=-1).astype(np.float32)
    sin = np.concatenate([np.sin(ah), np.sin(ah), np.sin(aw), np.sin(aw)],
                         axis=-1).astype(np.float32)
    cosp = np.tile(cos, (1, HEADS)).reshape(WIN, WIN, DIM)
    sinp = np.tile(sin, (1, HEADS)).reshape(WIN, WIN, DIM)
    return jnp.asarray(cosp), jnp.asarray(sinp)


_COSP, _SINP = _rope_tables()


def _router_body(x_ref, wr1_ref, wr2_ref, br1_ref, br2_ref, routes_ref,
                 prob_ref):
    x = x_ref[...]
    hdn = jnp.maximum(jnp.dot(x, wr1_ref[...],
                              preferred_element_type=jnp.float32)
                      + br1_ref[0, 0], 0.0)
    logits = jnp.dot(hdn, wr2_ref[...],
                     preferred_element_type=jnp.float32) + br2_ref[0, 0]
    m = jnp.max(logits, axis=-1, keepdims=True)
    s = jnp.sum(jnp.exp(logits - m), axis=-1, keepdims=True)
    prob = 1.0 / s
    idx = lax.broadcasted_iota(jnp.int32, logits.shape, 1)
    routes = jnp.min(jnp.where(logits == m, idx, NE), axis=-1, keepdims=True)
    routes_ref[...] = routes
    prob_ref[...] = prob


def _run_router(xf, Wr1, Wr2, br1, br2):
    return pl.pallas_call(
        _router_body,
        grid=(NTB,),
        in_specs=[
            pl.BlockSpec((TBLK, DIM), lambda t: (t, 0)),
            pl.BlockSpec((DIM, RHID), lambda t: (0, 0)),
            pl.BlockSpec((RHID, NE), lambda t: (0, 0)),
            pl.BlockSpec((1, 1, RHID), lambda t: (0, 0, 0)),
            pl.BlockSpec((1, 1, NE), lambda t: (0, 0, 0)),
        ],
        out_specs=[
            pl.BlockSpec((TBLK, 1), lambda t: (t, 0)),
            pl.BlockSpec((TBLK, 1), lambda t: (t, 0)),
        ],
        out_shape=[
            jax.ShapeDtypeStruct((NTOK, 1), jnp.int32),
            jax.ShapeDtypeStruct((NTOK, 1), jnp.float32),
        ],
    )(xf, Wr1, Wr2, br1.reshape(1, 1, RHID), br2.reshape(1, 1, NE))


def _moe_dense_body(x_ref, w_ref, b_ref, routes_ref, o_ref):
    e = pl.program_id(1)
    x = x_ref[...]
    contrib = jnp.dot(x, w_ref[0], preferred_element_type=jnp.float32)
    contrib = contrib + b_ref[0]
    mask = (routes_ref[...] == e)
    masked = jnp.where(mask, contrib, 0.0)

    @pl.when(e == 0)
    def _():
        o_ref[...] = masked

    @pl.when(e != 0)
    def _():
        o_ref[...] = o_ref[...] + masked


def _run_moe_dense(xf, W, b, routes, dout):
    return pl.pallas_call(
        _moe_dense_body,
        grid=(NTB, NE),
        in_specs=[
            pl.BlockSpec((TBLK, DIM), lambda t, e: (t, 0)),
            pl.BlockSpec((1, DIM, dout), lambda t, e: (e, 0, 0)),
            pl.BlockSpec((1, 1, dout), lambda t, e: (e, 0, 0)),
            pl.BlockSpec((TBLK, 1), lambda t, e: (t, 0)),
        ],
        out_specs=pl.BlockSpec((TBLK, dout), lambda t, e: (t, 0)),
        out_shape=jax.ShapeDtypeStruct((NTOK, dout), jnp.float32),
    )(xf, W, b.reshape(NE, 1, dout), routes)


def _rot_half_packed(x):
    xr = x.reshape(WIN * WIN, HEADS, HD)
    half = HD // 4
    x1 = xr[:, :, 0:half]
    x2 = xr[:, :, half:2 * half]
    x3 = xr[:, :, 2 * half:3 * half]
    x4 = xr[:, :, 3 * half:]
    out = jnp.concatenate([-x2, x1, -x4, x3], axis=-1)
    return out.reshape(WIN * WIN, DIM)


def _attn_body(qkv_ref, cos_ref, sin_ref, prob_ref, o_ref):
    qkv = qkv_ref[0].reshape(WIN * WIN, 3 * DIM)
    cos = cos_ref[...].reshape(WIN * WIN, DIM)
    sin = sin_ref[...].reshape(WIN * WIN, DIM)
    q = qkv[:, 0:DIM]
    k = qkv[:, DIM:2 * DIM]
    v = qkv[:, 2 * DIM:3 * DIM]
    q = q * cos + _rot_half_packed(q) * sin
    k = k * cos + _rot_half_packed(k) * sin
    outs = []
    for h in range(HEADS):
        qh = q[:, h * HD:(h + 1) * HD] * SCALE
        kh = k[:, h * HD:(h + 1) * HD]
        vh = v[:, h * HD:(h + 1) * HD]
        a = lax.dot_general(qh, kh, (((1,), (1,)), ((), ())),
                            preferred_element_type=jnp.float32)
        a = a - jnp.max(a, axis=-1, keepdims=True)
        p = jnp.exp(a)
        p = p / jnp.sum(p, axis=-1, keepdims=True)
        outs.append(jnp.dot(p, vh, preferred_element_type=jnp.float32))
    out = jnp.concatenate(outs, axis=-1)
    out = out * prob_ref[0].reshape(WIN * WIN, 1)
    o_ref[0] = out.reshape(WIN, WIN, DIM)


def _run_attention(qkv, prob, Bs, H, W):
    qkv4 = qkv.reshape(Bs, H, W, 3 * DIM)
    prob4 = prob.reshape(Bs, H, W, 1)
    return pl.pallas_call(
        _attn_body,
        grid=(Bs, H // WIN, W // WIN),
        in_specs=[
            pl.BlockSpec((1, WIN, WIN, 3 * DIM), lambda b, i, j: (b, i, j, 0)),
            pl.BlockSpec((WIN, WIN, DIM), lambda b, i, j: (0, 0, 0)),
            pl.BlockSpec((WIN, WIN, DIM), lambda b, i, j: (0, 0, 0)),
            pl.BlockSpec((1, WIN, WIN, 1), lambda b, i, j: (b, i, j, 0)),
        ],
        out_specs=pl.BlockSpec((1, WIN, WIN, DIM), lambda b, i, j: (b, i, j, 0)),
        out_shape=jax.ShapeDtypeStruct((Bs, H, W, DIM), jnp.float32),
    )(qkv4, _COSP, _SINP, prob4)


def kernel(x, Wqkv, bqkv, Wproj, bproj, Wr1, br1, Wr2, br2):
    Bs, H, W, C = x.shape
    xf = x.reshape(NTOK, C)
    routes, prob = _run_router(xf, Wr1, Wr2, br1, br2)
    qkv = _run_moe_dense(xf, Wqkv, bqkv, routes, 3 * DIM)
    attn = _run_attention(qkv, prob, Bs, H, W)
    attnf = attn.reshape(NTOK, DIM)
    proj = _run_moe_dense(attnf, Wproj, bproj, routes, DIM)
    return proj.reshape(Bs, H, W, C)


# speedup vs baseline: 1.1666x; 1.1666x over previous
import functools

import jax
import jax.numpy as jnp
import numpy as np
from jax import lax
from jax.experimental import pallas as pl
from jax.experimental.pallas import tpu as pltpu
from jax.experimental.pallas import tpu_sc as plsc

DIM = 256
HEADS = 8
HD = DIM // HEADS
WIN = 8
NE = 8
SCALE = HD ** -0.5
RHID = 128

NTOK = 2 * 64 * 64
TBLK = 512
NTB = NTOK // TBLK


def _rope_tables():
    d = HD // 2
    half = d // 2
    inv = 1.0 / (10000.0 ** (np.arange(half, dtype=np.float64) / half))
    hpos = np.repeat(np.arange(WIN), WIN).astype(np.float64)
    wpos = np.tile(np.arange(WIN), WIN).astype(np.float64)
    ah = hpos[:, None] * inv[None, :]
    aw = wpos[:, None] * inv[None, :]
    cos = np.concatenate([np.cos(ah), np.cos(ah), np.cos(aw), np.cos(aw)],
                         axis=-1).astype(np.float32)
    sin = np.concatenate([np.sin(ah), np.sin(ah), np.sin(aw), np.sin(aw)],
                         axis=-1).astype(np.float32)
    cosp = np.tile(cos, (1, HEADS)).reshape(WIN, WIN, DIM)
    sinp = np.tile(sin, (1, HEADS)).reshape(WIN, WIN, DIM)
    return jnp.asarray(cosp), jnp.asarray(sinp)


_COSP, _SINP = _rope_tables()


def _router_body(x_ref, wr1_ref, wr2_ref, br1_ref, br2_ref, routes_ref,
                 prob_ref):
    x = x_ref[...]
    hdn = jnp.maximum(jnp.dot(x, wr1_ref[...],
                              preferred_element_type=jnp.float32)
                      + br1_ref[0, 0], 0.0)
    logits = jnp.dot(hdn, wr2_ref[...],
                     preferred_element_type=jnp.float32) + br2_ref[0, 0]
    m = jnp.max(logits, axis=-1, keepdims=True)
    s = jnp.sum(jnp.exp(logits - m), axis=-1, keepdims=True)
    prob = 1.0 / s
    idx = lax.broadcasted_iota(jnp.int32, logits.shape, 1)
    routes = jnp.min(jnp.where(logits == m, idx, NE), axis=-1, keepdims=True)
    routes_ref[...] = routes
    prob_ref[...] = prob


def _run_router(xf, Wr1, Wr2, br1, br2):
    return pl.pallas_call(
        _router_body,
        grid=(NTB,),
        in_specs=[
            pl.BlockSpec((TBLK, DIM), lambda t: (t, 0)),
            pl.BlockSpec((DIM, RHID), lambda t: (0, 0)),
            pl.BlockSpec((RHID, NE), lambda t: (0, 0)),
            pl.BlockSpec((1, 1, RHID), lambda t: (0, 0, 0)),
            pl.BlockSpec((1, 1, NE), lambda t: (0, 0, 0)),
        ],
        out_specs=[
            pl.BlockSpec((TBLK, 1), lambda t: (t, 0)),
            pl.BlockSpec((TBLK, 1), lambda t: (t, 0)),
        ],
        out_shape=[
            jax.ShapeDtypeStruct((NTOK, 1), jnp.int32),
            jax.ShapeDtypeStruct((NTOK, 1), jnp.float32),
        ],
    )(xf, Wr1, Wr2, br1.reshape(1, 1, RHID), br2.reshape(1, 1, NE))


def _moe_dense_body(x_ref, w_ref, b_ref, routes_ref, o_ref):
    e = pl.program_id(1)
    x = x_ref[...]
    contrib = jnp.dot(x, w_ref[0], preferred_element_type=jnp.float32)
    contrib = contrib + b_ref[0]
    mask = (routes_ref[...] == e)
    masked = jnp.where(mask, contrib, 0.0)

    @pl.when(e == 0)
    def _():
        o_ref[...] = masked

    @pl.when(e != 0)
    def _():
        o_ref[...] = o_ref[...] + masked


def _run_moe_dense(xf, W, b, routes, dout):
    return pl.pallas_call(
        _moe_dense_body,
        grid=(NTB, NE),
        in_specs=[
            pl.BlockSpec((TBLK, DIM), lambda t, e: (t, 0)),
            pl.BlockSpec((1, DIM, dout), lambda t, e: (e, 0, 0)),
            pl.BlockSpec((1, 1, dout), lambda t, e: (e, 0, 0)),
            pl.BlockSpec((TBLK, 1), lambda t, e: (t, 0)),
        ],
        out_specs=pl.BlockSpec((TBLK, dout), lambda t, e: (t, 0)),
        out_shape=jax.ShapeDtypeStruct((NTOK, dout), jnp.float32),
    )(xf, W, b.reshape(NE, 1, dout), routes)


def _rot_half_packed(x):
    xr = x.reshape(WIN * WIN, HEADS, HD)
    half = HD // 4
    x1 = xr[:, :, 0:half]
    x2 = xr[:, :, half:2 * half]
    x3 = xr[:, :, 2 * half:3 * half]
    x4 = xr[:, :, 3 * half:]
    out = jnp.concatenate([-x2, x1, -x4, x3], axis=-1)
    return out.reshape(WIN * WIN, DIM)


def _attn_body(qkv_ref, cos_ref, sin_ref, prob_ref, o_ref):
    qkv = qkv_ref[0].reshape(WIN * WIN, 3 * DIM)
    cos = cos_ref[...].reshape(WIN * WIN, DIM)
    sin = sin_ref[...].reshape(WIN * WIN, DIM)
    q = qkv[:, 0:DIM]
    k = qkv[:, DIM:2 * DIM]
    v = qkv[:, 2 * DIM:3 * DIM]
    q = q * cos + _rot_half_packed(q) * sin
    k = k * cos + _rot_half_packed(k) * sin
    outs = []
    for h in range(HEADS):
        qh = q[:, h * HD:(h + 1) * HD] * SCALE
        kh = k[:, h * HD:(h + 1) * HD]
        vh = v[:, h * HD:(h + 1) * HD]
        a = lax.dot_general(qh, kh, (((1,), (1,)), ((), ())),
                            preferred_element_type=jnp.float32)
        a = a - jnp.max(a, axis=-1, keepdims=True)
        p = jnp.exp(a)
        p = p / jnp.sum(p, axis=-1, keepdims=True)
        outs.append(jnp.dot(p, vh, preferred_element_type=jnp.float32))
    out = jnp.concatenate(outs, axis=-1)
    out = out * prob_ref[0].reshape(WIN * WIN, 1)
    o_ref[0] = out.reshape(WIN, WIN, DIM)


def _run_attention(qkv, prob, Bs, H, W):
    qkv4 = qkv.reshape(Bs, H, W, 3 * DIM)
    prob4 = prob.reshape(Bs, H, W, 1)
    return pl.pallas_call(
        _attn_body,
        grid=(Bs, H // WIN, W // WIN),
        in_specs=[
            pl.BlockSpec((1, WIN, WIN, 3 * DIM), lambda b, i, j: (b, i, j, 0)),
            pl.BlockSpec((WIN, WIN, DIM), lambda b, i, j: (0, 0, 0)),
            pl.BlockSpec((WIN, WIN, DIM), lambda b, i, j: (0, 0, 0)),
            pl.BlockSpec((1, WIN, WIN, 1), lambda b, i, j: (b, i, j, 0)),
        ],
        out_specs=pl.BlockSpec((1, WIN, WIN, DIM), lambda b, i, j: (b, i, j, 0)),
        out_shape=jax.ShapeDtypeStruct((Bs, H, W, DIM), jnp.float32),
    )(qkv4, _COSP, _SINP, prob4)


BLK = 512
PAD = NTOK + NE * BLK
NB = PAD // BLK

_TU128 = jnp.asarray(np.triu(np.ones((128, 128), np.float32), 1))
_TL64 = jnp.asarray(np.tril(np.ones((64, 64), np.float32), -1))


def _dispatch_body(routes_ref, tu_ref, tl_ref, pos_ref, blkexp_ref):
    routes = routes_ref[...]
    tu = tu_ref[...]
    tl = tl_ref[...]
    pos = jnp.zeros(routes.shape, jnp.float32)
    blkexp = jnp.zeros((1, 128), jnp.float32)
    biota = (lax.broadcasted_iota(jnp.int32, (1, 128), 1) * BLK
             ).astype(jnp.float32)
    pad_off = jnp.float32(0.0)
    for e in range(NE):
        m = (routes == e).astype(jnp.float32)
        prefix = jnp.dot(m, tu, preferred_element_type=jnp.float32)
        tot = jnp.sum(m, axis=1, keepdims=True)
        rowcum = jnp.dot(tl, tot, preferred_element_type=jnp.float32)
        cnt = jnp.sum(tot)
        padded = jnp.ceil(cnt / BLK) * BLK
        pos = pos + m * (pad_off + rowcum + prefix)
        pad_off = pad_off + padded
        blkexp = blkexp + (biota >= pad_off).astype(jnp.float32)
    pos_ref[...] = pos.astype(jnp.int32)
    blkexp_ref[...] = jnp.minimum(blkexp, NE - 1).astype(jnp.int32)


def _run_dispatch(routes2d):
    return pl.pallas_call(
        _dispatch_body,
        out_shape=[
            jax.ShapeDtypeStruct((NTOK // 128, 128), jnp.int32),
            jax.ShapeDtypeStruct((1, 128), jnp.int32),
        ],
    )(routes2d, _TU128, _TL64)


def _sc_permute_rows(src, idx, nrows_out, d, gather):
    info = plsc.get_sparse_core_info()
    nc, ns = info.num_cores, info.num_subcores
    nw = nc * ns
    rows_per = NTOK // nw
    ch = 128
    nch = rows_per // ch
    mesh = plsc.VectorSubcoreMesh(core_axis_name="c", subcore_axis_name="s")

    @functools.partial(
        pl.kernel, mesh=mesh,
        out_type=jax.ShapeDtypeStruct((nrows_out, d), jnp.float32),
        scratch_types=[
            pltpu.VMEM((ch,), jnp.int32),
            pltpu.VMEM((ch, d), jnp.float32),
            pltpu.SemaphoreType.DMA,
        ],
    )
    def k(src_hbm, idx_hbm, out_hbm, idx_v, rows_v, sem):
        wid = lax.axis_index("s") * nc + lax.axis_index("c")
        base = wid * rows_per
        for c in range(nch):
            off = base + c * ch
            pltpu.sync_copy(idx_hbm.at[pl.ds(off, ch)], idx_v)
            if gather:
                pltpu.async_copy(src_hbm.at[idx_v], rows_v, sem).wait()
                pltpu.sync_copy(rows_v, out_hbm.at[pl.ds(off, ch)])
            else:
                pltpu.sync_copy(src_hbm.at[pl.ds(off, ch)], rows_v)
                pltpu.async_copy(rows_v, out_hbm.at[idx_v], sem).wait()

    return k(src, idx)


def _sc_scatter_rows(src, idx, nrows_out):
    return _sc_permute_rows(src, idx, nrows_out, src.shape[1], gather=False)


def _sc_gather_rows(src, idx):
    return _sc_permute_rows(src, idx, NTOK, src.shape[1], gather=True)


def _grouped_body(blkexp_sref, x_ref, w_ref, b_ref, o_ref):
    o_ref[...] = (jnp.dot(x_ref[...], w_ref[0],
                          preferred_element_type=jnp.float32) + b_ref[0])


def _run_grouped(xs, W, b, blkexp, dout):
    spec = pltpu.PrefetchScalarGridSpec(
        num_scalar_prefetch=1,
        grid=(NB,),
        in_specs=[
            pl.BlockSpec((BLK, DIM), lambda i, bref: (i, 0)),
            pl.BlockSpec((1, DIM, dout), lambda i, bref: (bref[i], 0, 0)),
            pl.BlockSpec((1, 1, dout), lambda i, bref: (bref[i], 0, 0)),
        ],
        out_specs=pl.BlockSpec((BLK, dout), lambda i, bref: (i, 0)),
    )
    return pl.pallas_call(
        _grouped_body,
        grid_spec=spec,
        out_shape=jax.ShapeDtypeStruct((PAD, dout), jnp.float32),
    )(blkexp, xs, W, b.reshape(NE, 1, dout))


def kernel(x, Wqkv, bqkv, Wproj, bproj, Wr1, br1, Wr2, br2):
    Bs, H, W, C = x.shape
    xf = x.reshape(NTOK, C)
    routes, prob = _run_router(xf, Wr1, Wr2, br1, br2)
    pos2d, blkexp2d = _run_dispatch(routes.reshape(NTOK // 128, 128))
    pos = pos2d.reshape(NTOK)
    blkexp = blkexp2d.reshape(128)
    xs = _sc_scatter_rows(xf, pos, PAD)
    qkvs = _run_grouped(xs, Wqkv, bqkv, blkexp, 3 * DIM)
    qkvt = _sc_gather_rows(qkvs, pos)
    attn = _run_attention(qkvt, prob, Bs, H, W)
    attnf = attn.reshape(NTOK, DIM)
    asrt = _sc_scatter_rows(attnf, pos, PAD)
    ps = _run_grouped(asrt, Wproj, bproj, blkexp, DIM)
    proj = _sc_gather_rows(ps, pos)
    return proj.reshape(Bs, H, W, C)


def _kernel_v1(x, Wqkv, bqkv, Wproj, bproj, Wr1, br1, Wr2, br2):
    Bs, H, W, C = x.shape
    xf = x.reshape(NTOK, C)
    routes, prob = _run_router(xf, Wr1, Wr2, br1, br2)
    qkv = _run_moe_dense(xf, Wqkv, bqkv, routes, 3 * DIM)
    attn = _run_attention(qkv, prob, Bs, H, W)
    attnf = attn.reshape(NTOK, DIM)
    proj = _run_moe_dense(attnf, Wproj, bproj, routes, DIM)
    return proj.reshape(Bs, H, W, C)
